# baseline (speedup 1.0000x reference)
import jax
import jax.numpy as jnp
from jax.experimental import pallas as pl
from jax.experimental.pallas import tpu as pltpu

N_TOKENS = 16384
D_MODEL = 2048
N_EXPERTS = 16
K = 2
BLOCK_T = 1024


def _router_body(x_ref, w_ref, gates_ref, vals_ref, inds_ref):
    x = x_ref[...]
    w = w_ref[...]
    logits = jnp.dot(x, w, preferred_element_type=jnp.float32)

    m = jnp.max(logits, axis=-1, keepdims=True)
    e = jnp.exp(logits - m)
    s = jnp.sum(e, axis=-1, keepdims=True)
    gates = e / s
    gates_ref[...] = gates

    iota = jax.lax.broadcasted_iota(jnp.int32, gates.shape, 1)
    m1 = jnp.max(gates, axis=-1, keepdims=True)
    i1 = jnp.min(jnp.where(gates == m1, iota, N_EXPERTS), axis=-1, keepdims=True)
    g2 = jnp.where(iota == i1, -jnp.inf, gates)
    m2 = jnp.max(g2, axis=-1, keepdims=True)
    i2 = jnp.min(jnp.where(g2 == m2, iota, N_EXPERTS), axis=-1, keepdims=True)

    vals_ref[...] = jnp.concatenate([m1, m2], axis=-1)
    inds_ref[...] = jnp.concatenate([i1, i2], axis=-1)


def kernel(hidden_states, gate_w, noise_w):
    del noise_w
    w_t = gate_w.T

    grid = (N_TOKENS // BLOCK_T,)
    gates, vals, inds = pl.pallas_call(
        _router_body,
        grid=grid,
        in_specs=[
            pl.BlockSpec((BLOCK_T, D_MODEL), lambda i: (i, 0)),
            pl.BlockSpec((D_MODEL, N_EXPERTS), lambda i: (0, 0)),
        ],
        out_specs=[
            pl.BlockSpec((BLOCK_T, N_EXPERTS), lambda i: (i, 0)),
            pl.BlockSpec((BLOCK_T, K), lambda i: (i, 0)),
            pl.BlockSpec((BLOCK_T, K), lambda i: (i, 0)),
        ],
        out_shape=[
            jax.ShapeDtypeStruct((N_TOKENS, N_EXPERTS), jnp.float32),
            jax.ShapeDtypeStruct((N_TOKENS, K), jnp.float32),
            jax.ShapeDtypeStruct((N_TOKENS, K), jnp.int32),
        ],
    )(hidden_states, w_t)
    return vals, inds, gates


# speedup vs baseline: 1.1097x; 1.1097x over previous
import jax
import jax.numpy as jnp
from jax.experimental import pallas as pl
from jax.experimental.pallas import tpu as pltpu

N_TOKENS = 16384
D_MODEL = 2048
N_EXPERTS = 16
K = 2
BLOCK_T = 1024


def _router_body(x_ref, w_ref, gates_ref, vals_ref, inds_ref):
    x = x_ref[...]
    w = w_ref[...]
    logits_t = jax.lax.dot_general(
        w, x, (((1,), (1,)), ((), ())), preferred_element_type=jnp.float32)

    m = jnp.max(logits_t, axis=0, keepdims=True)
    e = jnp.exp(logits_t - m)
    s = jnp.sum(e, axis=0, keepdims=True)
    gates_t = e / s
    gates_ref[...] = gates_t.T

    iota = jax.lax.broadcasted_iota(jnp.int32, gates_t.shape, 0)
    m1 = jnp.max(gates_t, axis=0, keepdims=True)
    i1 = jnp.min(jnp.where(gates_t == m1, iota, N_EXPERTS), axis=0, keepdims=True)
    g2 = jnp.where(iota == i1, -jnp.inf, gates_t)
    m2 = jnp.max(g2, axis=0, keepdims=True)
    i2 = jnp.min(jnp.where(g2 == m2, iota, N_EXPERTS), axis=0, keepdims=True)

    vals_ref[...] = jnp.concatenate([m1, m2], axis=0).T
    inds_ref[...] = jnp.concatenate([i1, i2], axis=0).T


def kernel(hidden_states, gate_w, noise_w):
    del noise_w

    grid = (N_TOKENS // BLOCK_T,)
    gates, vals, inds = pl.pallas_call(
        _router_body,
        grid=grid,
        in_specs=[
            pl.BlockSpec((BLOCK_T, D_MODEL), lambda i: (i, 0)),
            pl.BlockSpec((N_EXPERTS, D_MODEL), lambda i: (0, 0)),
        ],
        out_specs=[
            pl.BlockSpec((BLOCK_T, N_EXPERTS), lambda i: (i, 0)),
            pl.BlockSpec((BLOCK_T, K), lambda i: (i, 0)),
            pl.BlockSpec((BLOCK_T, K), lambda i: (i, 0)),
        ],
        out_shape=[
            jax.ShapeDtypeStruct((N_TOKENS, N_EXPERTS), jnp.float32),
            jax.ShapeDtypeStruct((N_TOKENS, K), jnp.float32),
            jax.ShapeDtypeStruct((N_TOKENS, K), jnp.int32),
        ],
    )(hidden_states, gate_w)
    return vals, inds, gates
